# Initial kernel scaffold; baseline (speedup 1.0000x reference)
import jax
import jax.numpy as jnp
from jax.experimental import pallas as pl


def kernel(bag_of_units_ids, bag_of_units_values, player_embed, neutral_embed, player_dense_weight, neutral_dense_weight):
    raise NotImplementedError("write your pallas kernel here")


# speedup vs baseline: 322.5819x; 322.5819x over previous
import functools

import jax
import jax.numpy as jnp
from jax import lax
from jax.experimental import pallas as pl
from jax.experimental.pallas import tpu as pltpu
from jax.experimental.pallas import tpu_sc as plsc

B, C, OV, W, H = 128, 3, 4, 64, 64
PIX = W * H
ITEM = OV * PIX
NC, NS = 2, 16
NW = NC * NS
ITEMS = B * C
B_PER_W = B // NW
CH_PERM = (1, 2, 0)
LANES = 16


def _sc_body(ids_hbm, vals_hbm, ptab_hbm, ntab_hbm, dw_hbm, out_hbm,
             ptab_v, ntab_v, dw_v, ids_v, vals_v, out_v,
             sem_i0, sem_i1, sem_v0, sem_v1, sem_o0, sem_o1):
  cid = lax.axis_index("c")
  sid = lax.axis_index("s")
  wid = sid * NC + cid

  pltpu.sync_copy(ptab_hbm, ptab_v)
  pltpu.sync_copy(ntab_hbm, ntab_v)
  pltpu.sync_copy(dw_hbm, dw_v)
  dwp = dw_v[pl.ds(0, LANES)]
  dwn = dw_v[pl.ds(LANES, LANES)]

  sem_i = [sem_i0, sem_i1]
  sem_v = [sem_v0, sem_v1]
  sem_o = [sem_o0, sem_o1]
  in_d = [None, None]
  val_d = [None, None]
  out_d = [None, None]

  def src_row(k):
    j, c = divmod(k, 3)
    return (wid * B_PER_W + j) * 3 + CH_PERM[c]

  def dst_row(k):
    return wid * (B_PER_W * 3) + k

  def start_in(k):
    buf = k % 2
    s = src_row(k)
    in_d[buf] = pltpu.async_copy(ids_hbm.at[s], ids_v.at[buf], sem_i[buf])
    val_d[buf] = pltpu.async_copy(vals_hbm.at[s], vals_v.at[buf], sem_v[buf])

  start_in(0)
  n_items = B_PER_W * 3
  for k in range(n_items):
    buf = k % 2
    if k + 1 < n_items:
      start_in(k + 1)
    in_d[buf].wait()
    val_d[buf].wait()
    if out_d[buf] is not None:
      out_d[buf].wait()

    c = k % 3
    tab = ntab_v if c == 1 else ptab_v
    dw = dwn if c == 1 else dwp

    def compute(i, _, buf=buf, tab=tab, dw=dw):
      off = i * LANES
      idx = ids_v[buf, pl.ds(off, LANES)]
      acc = plsc.load_gather(tab, [idx]) * vals_v[buf, pl.ds(off, LANES)]
      for ov in range(1, OV):
        o = ov * PIX + off
        idx = ids_v[buf, pl.ds(o, LANES)]
        acc = jnp.maximum(
            acc, plsc.load_gather(tab, [idx]) * vals_v[buf, pl.ds(o, LANES)])
      out_v[buf, pl.ds(off, LANES)] = acc * dw
      return _

    lax.fori_loop(0, PIX // LANES, compute, 0, unroll=2)

    out_d[buf] = pltpu.async_copy(out_v.at[buf], out_hbm.at[dst_row(k)],
                                  sem_o[buf])

  out_d[0].wait()
  out_d[1].wait()


_PTAB_PAD = 352
_NTAB_PAD = 112

@functools.cache
def _build_sc_call():
  return pl.kernel(
    _sc_body,
    out_type=jax.ShapeDtypeStruct((ITEMS, PIX), jnp.float32),
    mesh=plsc.VectorSubcoreMesh(
        core_axis_name="c", subcore_axis_name="s",
        num_cores=NC, num_subcores=NS),
    scratch_types=[
        pltpu.VMEM((_PTAB_PAD,), jnp.float32),
        pltpu.VMEM((_NTAB_PAD,), jnp.float32),
        pltpu.VMEM((2 * LANES,), jnp.float32),
        pltpu.VMEM((2, ITEM), jnp.int32),
        pltpu.VMEM((2, ITEM), jnp.float32),
        pltpu.VMEM((2, PIX), jnp.float32),
        pltpu.SemaphoreType.DMA,
        pltpu.SemaphoreType.DMA,
        pltpu.SemaphoreType.DMA,
        pltpu.SemaphoreType.DMA,
        pltpu.SemaphoreType.DMA,
        pltpu.SemaphoreType.DMA,
    ],
    compiler_params=pltpu.CompilerParams(needs_layout_passes=False),
  )


@jax.jit
def kernel(bag_of_units_ids, bag_of_units_values, player_embed, neutral_embed,
           player_dense_weight, neutral_dense_weight):
  ids = bag_of_units_ids.reshape(ITEMS, ITEM)
  vals = bag_of_units_values.reshape(ITEMS, ITEM)
  ptab = jnp.pad(player_embed.reshape(-1), (0, _PTAB_PAD - player_embed.size))
  ntab = jnp.pad(neutral_embed.reshape(-1), (0, _NTAB_PAD - neutral_embed.size))
  dw = jnp.concatenate([
      jnp.broadcast_to(player_dense_weight, (LANES,)),
      jnp.broadcast_to(neutral_dense_weight, (LANES,)),
  ])
  out = _build_sc_call()(ids, vals, ptab, ntab, dw)
  return out.reshape(B, C, W, H)


# speedup vs baseline: 329.2939x; 1.0208x over previous
import functools

import jax
import jax.numpy as jnp
from jax import lax
from jax.experimental import pallas as pl
from jax.experimental.pallas import tpu as pltpu
from jax.experimental.pallas import tpu_sc as plsc

B, C, OV, W, H = 128, 3, 4, 64, 64
PIX = W * H
ITEM = OV * PIX
NC, NS = 2, 16
NW = NC * NS
ITEMS = B * C
B_PER_W = B // NW
CH_PERM = (1, 2, 0)
LANES = 16


def _sc_body(ids_hbm, vals_hbm, ptab_hbm, ntab_hbm, dw_hbm, out_hbm,
             ptab_v, ntab_v, dw_v, ids_v, vals_v, out_v,
             sem_i0, sem_i1, sem_v0, sem_v1, sem_o0, sem_o1):
  cid = lax.axis_index("c")
  sid = lax.axis_index("s")
  wid = sid * NC + cid

  pltpu.sync_copy(ptab_hbm, ptab_v)
  pltpu.sync_copy(ntab_hbm, ntab_v)
  pltpu.sync_copy(dw_hbm, dw_v)
  dwp = dw_v[pl.ds(0, LANES)]
  dwn = dw_v[pl.ds(LANES, LANES)]

  sem_i = [sem_i0, sem_i1]
  sem_v = [sem_v0, sem_v1]
  sem_o = [sem_o0, sem_o1]
  in_d = [None, None]
  val_d = [None, None]
  out_d = [None, None]

  def src_row(k):
    j, c = divmod(k, 3)
    return (wid * B_PER_W + j) * 3 + CH_PERM[c]

  def dst_row(k):
    return wid * (B_PER_W * 3) + k

  def start_in(k):
    buf = k % 2
    s = src_row(k)
    in_d[buf] = pltpu.async_copy(ids_hbm.at[s], ids_v.at[buf], sem_i[buf])
    val_d[buf] = pltpu.async_copy(vals_hbm.at[s], vals_v.at[buf], sem_v[buf])

  start_in(0)
  n_items = B_PER_W * 3
  for k in range(n_items):
    buf = k % 2
    if k + 1 < n_items:
      start_in(k + 1)
    in_d[buf].wait()
    val_d[buf].wait()
    if out_d[buf] is not None:
      out_d[buf].wait()

    c = k % 3
    tab = ntab_v if c == 1 else ptab_v
    dw = dwn if c == 1 else dwp

    def compute(i, _, buf=buf, tab=tab, dw=dw):
      off = i * LANES
      idx = ids_v[buf, pl.ds(off, LANES)]
      acc = plsc.load_gather(tab, [idx]) * vals_v[buf, pl.ds(off, LANES)]
      for ov in range(1, OV):
        o = ov * PIX + off
        idx = ids_v[buf, pl.ds(o, LANES)]
        acc = jnp.maximum(
            acc, plsc.load_gather(tab, [idx]) * vals_v[buf, pl.ds(o, LANES)])
      out_v[buf, pl.ds(off, LANES)] = acc * dw
      return _

    lax.fori_loop(0, PIX // LANES, compute, 0, unroll=8)

    out_d[buf] = pltpu.async_copy(out_v.at[buf], out_hbm.at[dst_row(k)],
                                  sem_o[buf])

  out_d[0].wait()
  out_d[1].wait()


_PTAB_PAD = 352
_NTAB_PAD = 112

@functools.cache
def _build_sc_call():
  return pl.kernel(
    _sc_body,
    out_type=jax.ShapeDtypeStruct((ITEMS, PIX), jnp.float32),
    mesh=plsc.VectorSubcoreMesh(
        core_axis_name="c", subcore_axis_name="s",
        num_cores=NC, num_subcores=NS),
    scratch_types=[
        pltpu.VMEM((_PTAB_PAD,), jnp.float32),
        pltpu.VMEM((_NTAB_PAD,), jnp.float32),
        pltpu.VMEM((2 * LANES,), jnp.float32),
        pltpu.VMEM((2, ITEM), jnp.int32),
        pltpu.VMEM((2, ITEM), jnp.float32),
        pltpu.VMEM((2, PIX), jnp.float32),
        pltpu.SemaphoreType.DMA,
        pltpu.SemaphoreType.DMA,
        pltpu.SemaphoreType.DMA,
        pltpu.SemaphoreType.DMA,
        pltpu.SemaphoreType.DMA,
        pltpu.SemaphoreType.DMA,
    ],
    compiler_params=pltpu.CompilerParams(needs_layout_passes=False),
  )


@jax.jit
def kernel(bag_of_units_ids, bag_of_units_values, player_embed, neutral_embed,
           player_dense_weight, neutral_dense_weight):
  ids = bag_of_units_ids.reshape(ITEMS, ITEM)
  vals = bag_of_units_values.reshape(ITEMS, ITEM)
  ptab = jnp.pad(player_embed.reshape(-1), (0, _PTAB_PAD - player_embed.size))
  ntab = jnp.pad(neutral_embed.reshape(-1), (0, _NTAB_PAD - neutral_embed.size))
  dw = jnp.concatenate([
      jnp.broadcast_to(player_dense_weight, (LANES,)),
      jnp.broadcast_to(neutral_dense_weight, (LANES,)),
  ])
  out = _build_sc_call()(ids, vals, ptab, ntab, dw)
  return out.reshape(B, C, W, H)


# speedup vs baseline: 364.6779x; 1.1075x over previous
import functools

import jax
import jax.numpy as jnp
from jax import lax
from jax.experimental import pallas as pl
from jax.experimental.pallas import tpu as pltpu
from jax.experimental.pallas import tpu_sc as plsc

B, C, OV, W, H = 128, 3, 4, 64, 64
WHALF = W // 2
NC, NS = 2, 16
NW = NC * NS
B_PER_W = B // NW
CH_PERM = (1, 2, 0)
LANES = 16
HGRP = H // LANES


def _sc_body(ids_hbm, vals_hbm, ptab_hbm, ntab_hbm, dw_hbm, out_hbm,
             ptab_v, ntab_v, dw_v, ids_v, vals_v, out_v,
             sem_i0, sem_i1, sem_v0, sem_v1, sem_o0, sem_o1):
  cid = lax.axis_index("c")
  sid = lax.axis_index("s")
  wid = sid * NC + cid

  pltpu.sync_copy(ptab_hbm, ptab_v)
  pltpu.sync_copy(ntab_hbm, ntab_v)
  pltpu.sync_copy(dw_hbm, dw_v)
  dwp = dw_v[pl.ds(0, LANES)]
  dwn = dw_v[pl.ds(LANES, LANES)]

  sem_i = [sem_i0, sem_i1]
  sem_v = [sem_v0, sem_v1]
  sem_o = [sem_o0, sem_o1]
  in_d = [None, None]
  val_d = [None, None]
  out_d = [None, None]

  n_stages = B_PER_W * 3 * 2

  def stage_coords(k):
    item, half = divmod(k, 2)
    j, c = divmod(item, 3)
    return j, c, half * WHALF

  def start_in(k):
    buf = k % 2
    j, c, w0 = stage_coords(k)
    b = wid * B_PER_W + j
    ch = CH_PERM[c]
    in_d[buf] = pltpu.async_copy(
        ids_hbm.at[b, ch, :, pl.ds(w0, WHALF), :], ids_v.at[buf], sem_i[buf])
    val_d[buf] = pltpu.async_copy(
        vals_hbm.at[b, ch, :, pl.ds(w0, WHALF), :], vals_v.at[buf], sem_v[buf])

  start_in(0)
  for k in range(n_stages):
    buf = k % 2
    if k + 1 < n_stages:
      start_in(k + 1)
    in_d[buf].wait()
    val_d[buf].wait()
    if out_d[buf] is not None:
      out_d[buf].wait()

    j, c, w0 = stage_coords(k)
    tab = ntab_v if c == 1 else ptab_v
    dw = dwn if c == 1 else dwp

    def compute(i, _, buf=buf, tab=tab, dw=dw):
      w = i >> 2
      h0 = (i & 3) * LANES
      idx = ids_v[buf, 0, w, pl.ds(h0, LANES)]
      acc = plsc.load_gather(tab, [idx]) * vals_v[buf, 0, w, pl.ds(h0, LANES)]
      for ov in range(1, OV):
        idx = ids_v[buf, ov, w, pl.ds(h0, LANES)]
        acc = jnp.maximum(
            acc,
            plsc.load_gather(tab, [idx]) * vals_v[buf, ov, w, pl.ds(h0, LANES)])
      out_v[buf, w, pl.ds(h0, LANES)] = acc * dw
      return _

    lax.fori_loop(0, WHALF * HGRP, compute, 0, unroll=8)

    b = wid * B_PER_W + j
    out_d[buf] = pltpu.async_copy(
        out_v.at[buf], out_hbm.at[b, c, pl.ds(w0, WHALF), :], sem_o[buf])

  out_d[0].wait()
  out_d[1].wait()


_PTAB_PAD = 352
_NTAB_PAD = 112


@functools.cache
def _build_sc_call():
  return pl.kernel(
    _sc_body,
    out_type=jax.ShapeDtypeStruct((B, C, W, H), jnp.float32),
    mesh=plsc.VectorSubcoreMesh(
        core_axis_name="c", subcore_axis_name="s",
        num_cores=NC, num_subcores=NS),
    scratch_types=[
        pltpu.VMEM((_PTAB_PAD,), jnp.float32),
        pltpu.VMEM((_NTAB_PAD,), jnp.float32),
        pltpu.VMEM((2 * LANES,), jnp.float32),
        pltpu.VMEM((2, OV, WHALF, H), jnp.int32),
        pltpu.VMEM((2, OV, WHALF, H), jnp.float32),
        pltpu.VMEM((2, WHALF, H), jnp.float32),
        pltpu.SemaphoreType.DMA,
        pltpu.SemaphoreType.DMA,
        pltpu.SemaphoreType.DMA,
        pltpu.SemaphoreType.DMA,
        pltpu.SemaphoreType.DMA,
        pltpu.SemaphoreType.DMA,
    ],
    compiler_params=pltpu.CompilerParams(
        needs_layout_passes=False, use_tc_tiling_on_sc=True),
  )


@jax.jit
def kernel(bag_of_units_ids, bag_of_units_values, player_embed, neutral_embed,
           player_dense_weight, neutral_dense_weight):
  ptab = jnp.pad(player_embed.reshape(-1), (0, _PTAB_PAD - player_embed.size))
  ntab = jnp.pad(neutral_embed.reshape(-1), (0, _NTAB_PAD - neutral_embed.size))
  dw = jnp.concatenate([
      jnp.broadcast_to(player_dense_weight, (LANES,)),
      jnp.broadcast_to(neutral_dense_weight, (LANES,)),
  ])
  return _build_sc_call()(bag_of_units_ids, bag_of_units_values, ptab, ntab, dw)
